# baseline (speedup 1.0000x reference)
import jax
import jax.numpy as jnp
from jax.experimental import pallas as pl
from jax.experimental.pallas import tpu as pltpu

N_R = 4096
N_RP1 = 4096
D_OUT = 32
BM = 256
NBUF = 4


def _elu(v):
    return jnp.where(v > 0, v, jnp.exp(jnp.minimum(v, 0.0)) - 1.0)


def _fused_kernel(x1_ref, x2_ref, w1_ref, w2_ref, n11_hbm, n21_hbm,
                  out_hbm, xw1_ref, xw2_ref):
    xw1_ref[...] = jnp.dot(x1_ref[...], w1_ref[...],
                           preferred_element_type=jnp.float32)
    xw2_ref[...] = jnp.dot(x2_ref[...], w2_ref[...],
                           preferred_element_type=jnp.float32)

    def body(n11_blk, n21_blk, out_blk):
        up = jnp.dot(n11_blk[...], xw1_ref[...],
                     preferred_element_type=jnp.float32)
        cob = jnp.dot(n21_blk[...], xw2_ref[...],
                      preferred_element_type=jnp.float32)
        out_blk[...] = _elu(up) + _elu(cob)

    pipeline = pltpu.emit_pipeline(
        body,
        grid=(N_R // BM,),
        in_specs=[
            pl.BlockSpec((BM, N_R), lambda i: (i, 0),
                         pipeline_mode=pl.Buffered(NBUF)),
            pl.BlockSpec((BM, N_RP1), lambda i: (i, 0),
                         pipeline_mode=pl.Buffered(NBUF)),
        ],
        out_specs=[
            pl.BlockSpec((BM, D_OUT), lambda i: (i, 0)),
        ],
    )
    pipeline(n11_hbm, n21_hbm, out_hbm)


def kernel(x_1, x_2, neighborhood_1_to_1, neighborhood_2_to_1, W1, W2):
    return pl.pallas_call(
        _fused_kernel,
        in_specs=[
            pl.BlockSpec(memory_space=pltpu.VMEM),
            pl.BlockSpec(memory_space=pltpu.VMEM),
            pl.BlockSpec(memory_space=pltpu.VMEM),
            pl.BlockSpec(memory_space=pltpu.VMEM),
            pl.BlockSpec(memory_space=pl.ANY),
            pl.BlockSpec(memory_space=pl.ANY),
        ],
        out_specs=pl.BlockSpec(memory_space=pl.ANY),
        out_shape=jax.ShapeDtypeStruct((N_R, D_OUT), jnp.float32),
        scratch_shapes=[
            pltpu.VMEM((N_R, D_OUT), jnp.float32),
            pltpu.VMEM((N_RP1, D_OUT), jnp.float32),
        ],
    )(x_1, x_2, W1, W2, neighborhood_1_to_1, neighborhood_2_to_1)


# speedup vs baseline: 1.0156x; 1.0156x over previous
import jax
import jax.numpy as jnp
from jax.experimental import pallas as pl
from jax.experimental.pallas import tpu as pltpu

N_R = 4096
N_RP1 = 4096
D_OUT = 32
BM = 256
NB = N_R // BM
NBUF = 4


def _elu(v):
    return jnp.where(v > 0, v, jnp.exp(jnp.minimum(v, 0.0)) - 1.0)


def _copy(src_hbm, blk, dst_buf, slot, sem):
    return pltpu.make_async_copy(
        src_hbm.at[pl.ds(blk * BM, BM), :], dst_buf.at[slot], sem.at[slot])


def _fused_kernel(x1_ref, x2_ref, w1_ref, w2_ref, n11_hbm, n21_hbm,
                  out_ref, xw1_ref, xw2_ref, buf11, buf21, sem11, sem21):
    for s in range(NBUF):
        _copy(n11_hbm, s, buf11, s, sem11).start()
        _copy(n21_hbm, s, buf21, s, sem21).start()

    xw1_ref[...] = jnp.dot(x1_ref[...], w1_ref[...],
                           preferred_element_type=jnp.float32)
    xw2_ref[...] = jnp.dot(x2_ref[...], w2_ref[...],
                           preferred_element_type=jnp.float32)

    for i in range(NB):
        slot = i % NBUF
        _copy(n11_hbm, i, buf11, slot, sem11).wait()
        _copy(n21_hbm, i, buf21, slot, sem21).wait()
        up = jnp.dot(buf11[slot], xw1_ref[...],
                     preferred_element_type=jnp.float32)
        cob = jnp.dot(buf21[slot], xw2_ref[...],
                      preferred_element_type=jnp.float32)
        out_ref[pl.ds(i * BM, BM), :] = _elu(up) + _elu(cob)
        nxt = i + NBUF
        if nxt < NB:
            _copy(n11_hbm, nxt, buf11, slot, sem11).start()
            _copy(n21_hbm, nxt, buf21, slot, sem21).start()


def kernel(x_1, x_2, neighborhood_1_to_1, neighborhood_2_to_1, W1, W2):
    return pl.pallas_call(
        _fused_kernel,
        in_specs=[
            pl.BlockSpec(memory_space=pltpu.VMEM),
            pl.BlockSpec(memory_space=pltpu.VMEM),
            pl.BlockSpec(memory_space=pltpu.VMEM),
            pl.BlockSpec(memory_space=pltpu.VMEM),
            pl.BlockSpec(memory_space=pl.ANY),
            pl.BlockSpec(memory_space=pl.ANY),
        ],
        out_specs=pl.BlockSpec(memory_space=pltpu.VMEM),
        out_shape=jax.ShapeDtypeStruct((N_R, D_OUT), jnp.float32),
        scratch_shapes=[
            pltpu.VMEM((N_R, D_OUT), jnp.float32),
            pltpu.VMEM((N_RP1, D_OUT), jnp.float32),
            pltpu.VMEM((NBUF, BM, N_R), jnp.float32),
            pltpu.VMEM((NBUF, BM, N_RP1), jnp.float32),
            pltpu.SemaphoreType.DMA((NBUF,)),
            pltpu.SemaphoreType.DMA((NBUF,)),
        ],
        compiler_params=pltpu.CompilerParams(
            vmem_limit_bytes=100 * 1024 * 1024,
        ),
    )(x_1, x_2, W1, W2, neighborhood_1_to_1, neighborhood_2_to_1)


# speedup vs baseline: 1.0414x; 1.0254x over previous
import jax
import jax.numpy as jnp
from jax.experimental import pallas as pl
from jax.experimental.pallas import tpu as pltpu

N_R = 4096
N_RP1 = 4096
D_OUT = 32
BM = 256


def _elu(v):
    return jnp.where(v > 0, v, jnp.exp(jnp.minimum(v, 0.0)) - 1.0)


def _fused_kernel(x1_ref, x2_ref, w1_ref, w2_ref, n11_ref, n21_ref,
                  out_ref, xw1_ref, xw2_ref):
    i = pl.program_id(0)

    @pl.when(i == 0)
    def _project():
        xw1_ref[...] = jnp.dot(x1_ref[...], w1_ref[...],
                               preferred_element_type=jnp.float32)
        xw2_ref[...] = jnp.dot(x2_ref[...], w2_ref[...],
                               preferred_element_type=jnp.float32)

    up = jnp.dot(n11_ref[...], xw1_ref[...],
                 preferred_element_type=jnp.float32)
    cob = jnp.dot(n21_ref[...], xw2_ref[...],
                  preferred_element_type=jnp.float32)
    out_ref[...] = _elu(up) + _elu(cob)


def kernel(x_1, x_2, neighborhood_1_to_1, neighborhood_2_to_1, W1, W2):
    grid = (N_R // BM,)
    return pl.pallas_call(
        _fused_kernel,
        grid=grid,
        in_specs=[
            pl.BlockSpec((N_R, x_1.shape[1]), lambda i: (0, 0)),
            pl.BlockSpec((N_RP1, x_2.shape[1]), lambda i: (0, 0)),
            pl.BlockSpec((x_1.shape[1], D_OUT), lambda i: (0, 0)),
            pl.BlockSpec((x_2.shape[1], D_OUT), lambda i: (0, 0)),
            pl.BlockSpec((BM, N_R), lambda i: (i, 0)),
            pl.BlockSpec((BM, N_RP1), lambda i: (i, 0)),
        ],
        out_specs=pl.BlockSpec((BM, D_OUT), lambda i: (i, 0)),
        out_shape=jax.ShapeDtypeStruct((N_R, D_OUT), jnp.float32),
        scratch_shapes=[
            pltpu.VMEM((N_R, D_OUT), jnp.float32),
            pltpu.VMEM((N_RP1, D_OUT), jnp.float32),
        ],
        compiler_params=pltpu.CompilerParams(
            dimension_semantics=("arbitrary",),
        ),
    )(x_1, x_2, W1, W2, neighborhood_1_to_1, neighborhood_2_to_1)


# speedup vs baseline: 1.0817x; 1.0387x over previous
import jax
import jax.numpy as jnp
from jax.experimental import pallas as pl
from jax.experimental.pallas import tpu as pltpu

N_R = 4096
N_RP1 = 4096
D_OUT = 32
BM = 256


def _elu(v):
    return jnp.where(v > 0, v, jnp.exp(jnp.minimum(v, 0.0)) - 1.0)


def _fused_kernel(n11_ref, n21_ref, x1_ref, x2_ref, w1_ref, w2_ref,
                  out_ref, xw1_ref, xw2_ref):
    i = pl.program_id(0)

    @pl.when(i == 0)
    def _project():
        xw1_ref[...] = jnp.dot(x1_ref[...], w1_ref[...],
                               preferred_element_type=jnp.float32)
        xw2_ref[...] = jnp.dot(x2_ref[...], w2_ref[...],
                               preferred_element_type=jnp.float32)

    up = jnp.dot(n11_ref[...], xw1_ref[...],
                 preferred_element_type=jnp.float32)
    cob = jnp.dot(n21_ref[...], xw2_ref[...],
                  preferred_element_type=jnp.float32)
    out_ref[...] = _elu(up) + _elu(cob)


def kernel(x_1, x_2, neighborhood_1_to_1, neighborhood_2_to_1, W1, W2):
    grid = (N_R // BM,)
    return pl.pallas_call(
        _fused_kernel,
        grid=grid,
        in_specs=[
            pl.BlockSpec((BM, N_R), lambda i: (i, 0)),
            pl.BlockSpec((BM, N_RP1), lambda i: (i, 0)),
            pl.BlockSpec((N_R, x_1.shape[1]), lambda i: (0, 0)),
            pl.BlockSpec((N_RP1, x_2.shape[1]), lambda i: (0, 0)),
            pl.BlockSpec((x_1.shape[1], D_OUT), lambda i: (0, 0)),
            pl.BlockSpec((x_2.shape[1], D_OUT), lambda i: (0, 0)),
        ],
        out_specs=pl.BlockSpec((BM, D_OUT), lambda i: (i, 0)),
        out_shape=jax.ShapeDtypeStruct((N_R, D_OUT), jnp.float32),
        scratch_shapes=[
            pltpu.VMEM((N_R, D_OUT), jnp.float32),
            pltpu.VMEM((N_RP1, D_OUT), jnp.float32),
        ],
        compiler_params=pltpu.CompilerParams(
            dimension_semantics=("arbitrary",),
        ),
    )(neighborhood_1_to_1, neighborhood_2_to_1, x_1, x_2, W1, W2)
